# baseline (speedup 1.0000x reference)
import jax
import jax.numpy as jnp
import numpy as np
from jax.experimental import pallas as pl
from jax.experimental.pallas import tpu as pltpu

_G = 64
_KV = 27


def _kernel_maps(coords):
    n = coords.shape[0]
    M = _G + 2
    c = coords.astype(jnp.int32) + 1
    keys = c[:, 0] * (M * M) + c[:, 1] * M + c[:, 2]
    order = jnp.argsort(keys)
    skeys = keys[order]
    offs = []
    for dx in (-1, 0, 1):
        for dy in (-1, 0, 1):
            for dz in (-1, 0, 1):
                offs.append(dx * M * M + dy * M + dz)
    offs = jnp.asarray(offs, jnp.int32)
    q = keys[None, :] + offs[:, None]
    pos = jnp.searchsorted(skeys, q)
    posc = jnp.clip(pos, 0, n - 1)
    found = skeys[posc] == q
    return jnp.where(found, order[posc], n)


def _mm_acc_body(xg_ref, w_ref, o_ref):
    k = pl.program_id(1)

    @pl.when(k == 0)
    def _():
        o_ref[...] = jnp.zeros_like(o_ref)

    o_ref[...] += jnp.dot(xg_ref[0], w_ref[0],
                          preferred_element_type=jnp.float32)


def _conv_mm(xg, W, tr):
    K, n, cin = xg.shape
    cout = W.shape[2]
    T = n // tr
    return pl.pallas_call(
        _mm_acc_body,
        grid=(T, K),
        in_specs=[
            pl.BlockSpec((1, tr, cin), lambda i, k: (k, i, 0)),
            pl.BlockSpec((1, cin, cout), lambda i, k: (k, 0, 0)),
        ],
        out_specs=pl.BlockSpec((tr, cout), lambda i, k: (i, 0)),
        out_shape=jax.ShapeDtypeStruct((n, cout), jnp.float32),
        compiler_params=pltpu.CompilerParams(
            dimension_semantics=("parallel", "arbitrary")),
    )(xg, W)


def kernel(x, coords, W1, W2, W3, W4):
    n = x.shape[0]
    nbr = _kernel_maps(coords)

    x_pad = jnp.concatenate([x, jnp.zeros((1, 2), x.dtype)])
    Xg = x_pad[nbr].transpose(1, 0, 2).reshape(n, 2 * _KV)
    Xg_pad = jnp.concatenate([Xg, jnp.zeros((1, 2 * _KV), Xg.dtype)])
    Xgg = Xg_pad[nbr]

    W1cat = W1.reshape(2 * _KV, 256)
    V2 = jnp.einsum('ac,kcd->kad', W1cat, W2,
                    precision=jax.lax.Precision.HIGHEST)
    h2 = _conv_mm(Xgg, V2, tr=10000)

    h2_pad = jnp.concatenate([h2, jnp.zeros((1, 256), h2.dtype)])
    H = h2_pad[nbr]

    W4cat = W4.transpose(1, 0, 2).reshape(256, 2 * _KV)
    U = jnp.einsum('kab,bc->kac', W3, W4cat,
                   precision=jax.lax.Precision.HIGHEST)
    y = _conv_mm(H, U, tr=10000)

    y_r = jnp.concatenate([y, jnp.zeros((1, 2 * _KV), y.dtype)])
    y_r = y_r.reshape(n + 1, _KV, 2)
    z = y_r[nbr, jnp.arange(_KV)[:, None]]
    return z.sum(axis=0)
